# initial kernel scaffold (unmeasured)
import jax
import jax.numpy as jnp
from jax import lax
from jax.experimental import pallas as pl
from jax.experimental.pallas import tpu as pltpu

N_DEV = 8
B, SQ, D = 2, 128, 512
DH = 64


def kernel(x, Wq, Wk, Wv, Wo):
    HL = Wq.shape[1] // DH
    HD = HL * DH

    def body(x_ref, wq_ref, wk_ref, wv_ref, wo_ref, out_ref,
             comm_ref, send_sems, recv_sems):
        my = lax.axis_index("i")
        left = (my + N_DEV - 1) % N_DEV
        right = (my + 1) % N_DEV

        lane = lax.broadcasted_iota(jnp.int32, (SQ, HD), 1)
        pos = lax.broadcasted_iota(jnp.float32, (SQ, HD), 0)
        k2 = ((lane % DH) // 2).astype(jnp.float32) * (2.0 / DH)
        ang = pos * jnp.exp(-jnp.log(10000.0) * k2)
        cosm = jnp.cos(ang)
        sinm = jnp.sin(ang)

        jrow = lax.broadcasted_iota(jnp.int32, (HD, HD), 0)
        lcol = lax.broadcasted_iota(jnp.int32, (HD, HD), 1)
        perm = jnp.where((lcol == jrow + 1) & (jrow % 2 == 0), 1.0, 0.0) + \
               jnp.where((lcol == jrow - 1) & (jrow % 2 == 1), -1.0, 0.0)
        perm = perm.astype(jnp.float32)

        def rope(t):
            t_r = jnp.dot(t, perm, preferred_element_type=jnp.float32)
            return t * cosm + t_r * sinm

        for b in range(B):
            xb = x_ref[b, :, :]
            q = rope(jnp.dot(xb, wq_ref[:, :], preferred_element_type=jnp.float32))
            k = rope(jnp.dot(xb, wk_ref[:, :], preferred_element_type=jnp.float32))
            v = jnp.dot(xb, wv_ref[:, :], preferred_element_type=jnp.float32)
            ctxs = []
            for h in range(HL):
                sl = slice(h * DH, (h + 1) * DH)
                s = jnp.dot(q[:, sl], k[:, sl].T,
                            preferred_element_type=jnp.float32) * 0.125
                s = s - jnp.max(s, axis=-1, keepdims=True)
                w = jnp.exp(s)
                w = w / jnp.sum(w, axis=-1, keepdims=True)
                ctxs.append(jnp.dot(w, v[:, sl],
                                    preferred_element_type=jnp.float32))
            ctx = jnp.concatenate(ctxs, axis=1)
            pb = jnp.dot(ctx, wo_ref[:, :], preferred_element_type=jnp.float32)
            out_ref[b, :, :] = pb
            comm_ref[0, b, :, :] = pb

        barrier = pltpu.get_barrier_semaphore()
        for nbr in (left, right):
            pl.semaphore_signal(barrier, inc=1, device_id=(nbr,),
                                device_id_type=pl.DeviceIdType.MESH)
        pl.semaphore_wait(barrier, 2)

        for h in range(N_DEV - 1):
            rdma = pltpu.make_async_remote_copy(
                src_ref=comm_ref.at[h],
                dst_ref=comm_ref.at[h + 1],
                send_sem=send_sems.at[h],
                recv_sem=recv_sems.at[h],
                device_id=(right,),
                device_id_type=pl.DeviceIdType.MESH,
            )
            rdma.start()
            rdma.wait()
            out_ref[:, :, :] = out_ref[:, :, :] + comm_ref[h + 1, :, :, :]

    return pl.pallas_call(
        body,
        out_shape=jax.ShapeDtypeStruct((B, SQ, D), jnp.float32),
        in_specs=[pl.BlockSpec(memory_space=pltpu.VMEM)] * 5,
        out_specs=pl.BlockSpec(memory_space=pltpu.VMEM),
        scratch_shapes=[
            pltpu.VMEM((N_DEV, B, SQ, D), jnp.float32),
            pltpu.SemaphoreType.DMA((N_DEV - 1,)),
            pltpu.SemaphoreType.DMA((N_DEV - 1,)),
        ],
        compiler_params=pltpu.CompilerParams(collective_id=0),
    )(x, Wq, Wk, Wv, Wo)


# baseline (device time: 66275 ns/iter reference)
import jax
import jax.numpy as jnp
from jax import lax
from jax.experimental import pallas as pl
from jax.experimental.pallas import tpu as pltpu

N_DEV = 8
B, SQ, D = 2, 128, 512
DH = 64


def kernel(x, Wq, Wk, Wv, Wo):
    HL = Wq.shape[1] // DH
    HD = HL * DH

    def body(x_ref, wq_ref, wk_ref, wv_ref, wo_ref, out_ref,
             comm_ref, send_sems, recv_sems):
        my = lax.axis_index("i")
        left = (my + N_DEV - 1) % N_DEV
        right = (my + 1) % N_DEV

        lane = lax.broadcasted_iota(jnp.int32, (SQ, HD), 1)
        pos = lax.broadcasted_iota(jnp.int32, (SQ, HD), 0).astype(jnp.float32)
        k2 = ((lane % DH) // 2).astype(jnp.float32) * (2.0 / DH)
        ang = pos * jnp.exp(-jnp.log(10000.0) * k2)
        cosm = jnp.cos(ang)
        sinm = jnp.sin(ang)

        jrow = lax.broadcasted_iota(jnp.int32, (HD, HD), 0)
        lcol = lax.broadcasted_iota(jnp.int32, (HD, HD), 1)
        perm = jnp.where((lcol == jrow + 1) & (jrow % 2 == 0), 1.0, 0.0) + \
               jnp.where((lcol == jrow - 1) & (jrow % 2 == 1), -1.0, 0.0)
        perm = perm.astype(jnp.float32)

        def rope(t):
            t_r = jnp.dot(t, perm, preferred_element_type=jnp.float32)
            return t * cosm + t_r * sinm

        for b in range(B):
            xb = x_ref[b, :, :]
            q = rope(jnp.dot(xb, wq_ref[:, :], preferred_element_type=jnp.float32))
            k = rope(jnp.dot(xb, wk_ref[:, :], preferred_element_type=jnp.float32))
            v = jnp.dot(xb, wv_ref[:, :], preferred_element_type=jnp.float32)
            ctxs = []
            for h in range(HL):
                sl = slice(h * DH, (h + 1) * DH)
                s = jnp.dot(q[:, sl], k[:, sl].T,
                            preferred_element_type=jnp.float32) * 0.125
                s = s - jnp.max(s, axis=-1, keepdims=True)
                w = jnp.exp(s)
                w = w / jnp.sum(w, axis=-1, keepdims=True)
                ctxs.append(jnp.dot(w, v[:, sl],
                                    preferred_element_type=jnp.float32))
            ctx = jnp.concatenate(ctxs, axis=1)
            pb = jnp.dot(ctx, wo_ref[:, :], preferred_element_type=jnp.float32)
            out_ref[b, :, :] = pb
            comm_ref[0, b, :, :] = pb

        barrier = pltpu.get_barrier_semaphore()
        for nbr in (left, right):
            pl.semaphore_signal(barrier, inc=1, device_id=(nbr,),
                                device_id_type=pl.DeviceIdType.MESH)
        pl.semaphore_wait(barrier, 2)

        for h in range(N_DEV - 1):
            rdma = pltpu.make_async_remote_copy(
                src_ref=comm_ref.at[h],
                dst_ref=comm_ref.at[h + 1],
                send_sem=send_sems.at[h],
                recv_sem=recv_sems.at[h],
                device_id=(right,),
                device_id_type=pl.DeviceIdType.MESH,
            )
            rdma.start()
            rdma.wait()
            out_ref[:, :, :] = out_ref[:, :, :] + comm_ref[h + 1, :, :, :]

    return pl.pallas_call(
        body,
        out_shape=jax.ShapeDtypeStruct((B, SQ, D), jnp.float32),
        in_specs=[pl.BlockSpec(memory_space=pltpu.VMEM)] * 5,
        out_specs=pl.BlockSpec(memory_space=pltpu.VMEM),
        scratch_shapes=[
            pltpu.VMEM((N_DEV, B, SQ, D), jnp.float32),
            pltpu.SemaphoreType.DMA((N_DEV - 1,)),
            pltpu.SemaphoreType.DMA((N_DEV - 1,)),
        ],
        compiler_params=pltpu.CompilerParams(collective_id=0),
    )(x, Wq, Wk, Wv, Wo)


# device time: 24901 ns/iter; 2.6615x vs baseline; 2.6615x over previous
import jax
import jax.numpy as jnp
from jax import lax
from jax.experimental import pallas as pl
from jax.experimental.pallas import tpu as pltpu

N_DEV = 8
B, SQ, D = 2, 128, 512
DH = 64
CH = SQ // N_DEV


def kernel(x, Wq, Wk, Wv, Wo):
    HL = Wq.shape[1] // DH
    HD = HL * DH

    def body(x_ref, wq_ref, wk_ref, wv_ref, wo_ref, out_ref,
             pch_ref, rbuf1_ref, rbuf2_ref, gsrc_ref,
             s1_sems, r1_sems, s2_sems, r2_sems):
        my = lax.axis_index("i")

        lane = lax.broadcasted_iota(jnp.int32, (B * SQ, HD), 1)
        pos = (lax.broadcasted_iota(jnp.int32, (B * SQ, HD), 0) % SQ
               ).astype(jnp.float32)
        k2 = ((lane % DH) // 2).astype(jnp.float32) * (2.0 / DH)
        ang = pos * jnp.exp(-jnp.log(10000.0) * k2)
        cosm = jnp.cos(ang)
        sinm = jnp.sin(ang)

        jrow = lax.broadcasted_iota(jnp.int32, (HD, HD), 0)
        lcol = lax.broadcasted_iota(jnp.int32, (HD, HD), 1)
        perm = (jnp.where((lcol == jrow + 1) & (jrow % 2 == 0), 1.0, 0.0) +
                jnp.where((lcol == jrow - 1) & (jrow % 2 == 1), -1.0, 0.0)
                ).astype(jnp.float32)

        def rope(t):
            t_r = jnp.dot(t, perm, preferred_element_type=jnp.float32)
            return t * cosm + t_r * sinm

        xb = x_ref[:, :, :].reshape(B * SQ, D)
        q = rope(jnp.dot(xb, wq_ref[:, :], preferred_element_type=jnp.float32))
        k = rope(jnp.dot(xb, wk_ref[:, :], preferred_element_type=jnp.float32))
        v = jnp.dot(xb, wv_ref[:, :], preferred_element_type=jnp.float32)
        ctx_rows = []
        for b in range(B):
            r = slice(b * SQ, (b + 1) * SQ)
            ctxs = []
            for h in range(HL):
                c = slice(h * DH, (h + 1) * DH)
                s = jnp.dot(q[r, c], k[r, c].T,
                            preferred_element_type=jnp.float32) * 0.125
                s = s - jnp.max(s, axis=-1, keepdims=True)
                w = jnp.exp(s)
                w = w / jnp.sum(w, axis=-1, keepdims=True)
                ctxs.append(jnp.dot(w, v[r, c],
                                    preferred_element_type=jnp.float32))
            ctx_rows.append(jnp.concatenate(ctxs, axis=1))
        ctx = jnp.concatenate(ctx_rows, axis=0)
        pb = jnp.dot(ctx, wo_ref[:, :], preferred_element_type=jnp.float32)
        for j in range(N_DEV):
            for b in range(B):
                pch_ref[j, b, :, :] = pb[b * SQ + j * CH: b * SQ + (j + 1) * CH, :]

        barrier = pltpu.get_barrier_semaphore()
        for t in range(N_DEV):
            @pl.when(t != my)
            def _(t=t):
                pl.semaphore_signal(barrier, inc=1, device_id=(t,),
                                    device_id_type=pl.DeviceIdType.MESH)
        pl.semaphore_wait(barrier, N_DEV - 1)

        for t in range(N_DEV):
            @pl.when(t != my)
            def _(t=t):
                rdma = pltpu.make_async_remote_copy(
                    src_ref=pch_ref.at[t],
                    dst_ref=rbuf1_ref.at[my],
                    send_sem=s1_sems.at[t],
                    recv_sem=r1_sems.at[my],
                    device_id=(t,),
                    device_id_type=pl.DeviceIdType.MESH,
                )
                rdma.start()
        rbuf1_ref[my] = pch_ref[my]
        for s in range(N_DEV):
            @pl.when(s != my)
            def _(s=s):
                recv = pltpu.make_async_remote_copy(
                    src_ref=pch_ref.at[s],
                    dst_ref=rbuf1_ref.at[s],
                    send_sem=s1_sems.at[s],
                    recv_sem=r1_sems.at[s],
                    device_id=(0,),
                    device_id_type=pl.DeviceIdType.MESH,
                )
                recv.wait_recv()
        acc = rbuf1_ref[0, :, :, :]
        for s in range(1, N_DEV):
            acc = acc + rbuf1_ref[s, :, :, :]
        gsrc_ref[:, :, :] = acc
        rbuf2_ref[my] = acc

        for t in range(N_DEV):
            @pl.when(t != my)
            def _(t=t):
                rdma = pltpu.make_async_remote_copy(
                    src_ref=gsrc_ref,
                    dst_ref=rbuf2_ref.at[my],
                    send_sem=s2_sems.at[t],
                    recv_sem=r2_sems.at[my],
                    device_id=(t,),
                    device_id_type=pl.DeviceIdType.MESH,
                )
                rdma.start()
        for s in range(N_DEV):
            @pl.when(s != my)
            def _(s=s):
                recv = pltpu.make_async_remote_copy(
                    src_ref=gsrc_ref,
                    dst_ref=rbuf2_ref.at[s],
                    send_sem=s2_sems.at[s],
                    recv_sem=r2_sems.at[s],
                    device_id=(0,),
                    device_id_type=pl.DeviceIdType.MESH,
                )
                recv.wait_recv()
            out_ref[:, s * CH:(s + 1) * CH, :] = rbuf2_ref[s, :, :, :]

        for t in range(N_DEV):
            @pl.when(t != my)
            def _(t=t):
                w1 = pltpu.make_async_remote_copy(
                    src_ref=pch_ref.at[t], dst_ref=rbuf1_ref.at[t],
                    send_sem=s1_sems.at[t], recv_sem=r1_sems.at[t],
                    device_id=(0,), device_id_type=pl.DeviceIdType.MESH,
                )
                w1.wait_send()
                w2 = pltpu.make_async_remote_copy(
                    src_ref=gsrc_ref, dst_ref=rbuf2_ref.at[t],
                    send_sem=s2_sems.at[t], recv_sem=r2_sems.at[t],
                    device_id=(0,), device_id_type=pl.DeviceIdType.MESH,
                )
                w2.wait_send()

    return pl.pallas_call(
        body,
        out_shape=jax.ShapeDtypeStruct((B, SQ, D), jnp.float32),
        in_specs=[pl.BlockSpec(memory_space=pltpu.VMEM)] * 5,
        out_specs=pl.BlockSpec(memory_space=pltpu.VMEM),
        scratch_shapes=[
            pltpu.VMEM((N_DEV, B, CH, D), jnp.float32),
            pltpu.VMEM((N_DEV, B, CH, D), jnp.float32),
            pltpu.VMEM((N_DEV, B, CH, D), jnp.float32),
            pltpu.VMEM((B, CH, D), jnp.float32),
            pltpu.SemaphoreType.DMA((N_DEV,)),
            pltpu.SemaphoreType.DMA((N_DEV,)),
            pltpu.SemaphoreType.DMA((N_DEV,)),
            pltpu.SemaphoreType.DMA((N_DEV,)),
        ],
        compiler_params=pltpu.CompilerParams(collective_id=0),
    )(x, Wq, Wk, Wv, Wo)


# device time: 23848 ns/iter; 2.7791x vs baseline; 1.0442x over previous
import numpy as np

import jax
import jax.numpy as jnp
from jax import lax
from jax.experimental import pallas as pl
from jax.experimental.pallas import tpu as pltpu

N_DEV = 8
B, SQ, D = 2, 128, 512
DH = 64
CH = SQ // N_DEV


def _rope_tables(HD):
    lane = np.arange(HD)
    inv = 10000.0 ** (-(2.0 * ((lane % DH) // 2)) / DH)
    ang = np.arange(SQ)[:, None] * inv[None, :]
    cos = np.tile(np.cos(ang), (B, 1)).astype(np.float32)
    sin = np.tile(np.sin(ang), (B, 1)).astype(np.float32)
    perm = np.zeros((HD, HD), np.float32)
    perm[np.arange(1, HD, 2), np.arange(0, HD, 2)] = -1.0
    perm[np.arange(0, HD, 2), np.arange(1, HD, 2)] = 1.0
    return cos, sin, perm


def kernel(x, Wq, Wk, Wv, Wo):
    HL = Wq.shape[1] // DH
    HD = HL * DH
    cos_np, sin_np, perm_np = _rope_tables(HD)

    def body(x_ref, wq_ref, wk_ref, wv_ref, wo_ref, cos_ref, sin_ref,
             perm_ref, out_ref,
             pch_ref, rbuf1_ref, rbuf2_ref, gsrc_ref,
             s1_sems, r1_sems, s2_sems, r2_sems):
        my = lax.axis_index("i")

        def rope(t):
            t_r = jnp.dot(t, perm_ref[:, :], preferred_element_type=jnp.float32)
            return t * cos_ref[:, :] + t_r * sin_ref[:, :]

        xb = x_ref[:, :, :].reshape(B * SQ, D)
        q = rope(jnp.dot(xb, wq_ref[:, :], preferred_element_type=jnp.float32))
        k = rope(jnp.dot(xb, wk_ref[:, :], preferred_element_type=jnp.float32))
        v = jnp.dot(xb, wv_ref[:, :], preferred_element_type=jnp.float32)
        ctx_rows = []
        for b in range(B):
            r = slice(b * SQ, (b + 1) * SQ)
            ctxs = []
            for h in range(HL):
                c = slice(h * DH, (h + 1) * DH)
                s = jnp.dot(q[r, c], k[r, c].T,
                            preferred_element_type=jnp.float32) * 0.125
                w = jnp.exp(s)
                w = w / jnp.sum(w, axis=-1, keepdims=True)
                ctxs.append(jnp.dot(w, v[r, c],
                                    preferred_element_type=jnp.float32))
            ctx_rows.append(jnp.concatenate(ctxs, axis=1))
        ctx = jnp.concatenate(ctx_rows, axis=0)

        barrier = pltpu.get_barrier_semaphore()
        for t in range(N_DEV):
            @pl.when(t != my)
            def _(t=t):
                pl.semaphore_signal(barrier, inc=1, device_id=(t,),
                                    device_id_type=pl.DeviceIdType.MESH)
        pb = jnp.dot(ctx, wo_ref[:, :], preferred_element_type=jnp.float32)
        pbc = pb.astype(jnp.bfloat16)
        pl.semaphore_wait(barrier, N_DEV - 1)

        for j in range(N_DEV):
            for b in range(B):
                pch_ref[j, b, :, :] = pbc[b * SQ + j * CH: b * SQ + (j + 1) * CH, :]
            @pl.when(j != my)
            def _(j=j):
                rdma = pltpu.make_async_remote_copy(
                    src_ref=pch_ref.at[j],
                    dst_ref=rbuf1_ref.at[my],
                    send_sem=s1_sems.at[j],
                    recv_sem=r1_sems.at[my],
                    device_id=(j,),
                    device_id_type=pl.DeviceIdType.MESH,
                )
                rdma.start()
        rbuf1_ref[my] = pch_ref[my]
        for s in range(N_DEV):
            @pl.when(s != my)
            def _(s=s):
                recv = pltpu.make_async_remote_copy(
                    src_ref=pch_ref.at[s],
                    dst_ref=rbuf1_ref.at[s],
                    send_sem=s1_sems.at[s],
                    recv_sem=r1_sems.at[s],
                    device_id=(0,),
                    device_id_type=pl.DeviceIdType.MESH,
                )
                recv.wait_recv()
        acc = rbuf1_ref[0, :, :, :].astype(jnp.float32)
        for s in range(1, N_DEV):
            acc = acc + rbuf1_ref[s, :, :, :].astype(jnp.float32)
        gsrc_ref[:, :, :] = acc.astype(jnp.bfloat16)
        rbuf2_ref[my] = gsrc_ref[:, :, :]

        for t in range(N_DEV):
            @pl.when(t != my)
            def _(t=t):
                rdma = pltpu.make_async_remote_copy(
                    src_ref=gsrc_ref,
                    dst_ref=rbuf2_ref.at[my],
                    send_sem=s2_sems.at[t],
                    recv_sem=r2_sems.at[my],
                    device_id=(t,),
                    device_id_type=pl.DeviceIdType.MESH,
                )
                rdma.start()
        for s in range(N_DEV):
            @pl.when(s != my)
            def _(s=s):
                recv = pltpu.make_async_remote_copy(
                    src_ref=gsrc_ref,
                    dst_ref=rbuf2_ref.at[s],
                    send_sem=s2_sems.at[s],
                    recv_sem=r2_sems.at[s],
                    device_id=(0,),
                    device_id_type=pl.DeviceIdType.MESH,
                )
                recv.wait_recv()
            out_ref[:, s * CH:(s + 1) * CH, :] = \
                rbuf2_ref[s, :, :, :].astype(jnp.float32)

        for t in range(N_DEV):
            @pl.when(t != my)
            def _(t=t):
                w1 = pltpu.make_async_remote_copy(
                    src_ref=pch_ref.at[t], dst_ref=rbuf1_ref.at[t],
                    send_sem=s1_sems.at[t], recv_sem=r1_sems.at[t],
                    device_id=(0,), device_id_type=pl.DeviceIdType.MESH,
                )
                w1.wait_send()
                w2 = pltpu.make_async_remote_copy(
                    src_ref=gsrc_ref, dst_ref=rbuf2_ref.at[t],
                    send_sem=s2_sems.at[t], recv_sem=r2_sems.at[t],
                    device_id=(0,), device_id_type=pl.DeviceIdType.MESH,
                )
                w2.wait_send()

    return pl.pallas_call(
        body,
        out_shape=jax.ShapeDtypeStruct((B, SQ, D), jnp.float32),
        in_specs=[pl.BlockSpec(memory_space=pltpu.VMEM)] * 8,
        out_specs=pl.BlockSpec(memory_space=pltpu.VMEM),
        scratch_shapes=[
            pltpu.VMEM((N_DEV, B, CH, D), jnp.bfloat16),
            pltpu.VMEM((N_DEV, B, CH, D), jnp.bfloat16),
            pltpu.VMEM((N_DEV, B, CH, D), jnp.bfloat16),
            pltpu.VMEM((B, CH, D), jnp.bfloat16),
            pltpu.SemaphoreType.DMA((N_DEV,)),
            pltpu.SemaphoreType.DMA((N_DEV,)),
            pltpu.SemaphoreType.DMA((N_DEV,)),
            pltpu.SemaphoreType.DMA((N_DEV,)),
        ],
        compiler_params=pltpu.CompilerParams(collective_id=0),
    )(x, Wq, Wk, Wv, Wo,
      jnp.asarray(cos_np), jnp.asarray(sin_np), jnp.asarray(perm_np))


# device time: 18278 ns/iter; 3.6259x vs baseline; 1.3047x over previous
import numpy as np

import jax
import jax.numpy as jnp
from jax import lax
from jax.experimental import pallas as pl
from jax.experimental.pallas import tpu as pltpu

N_DEV = 8
B, SQ, D = 2, 128, 512
DH = 64
R = B * SQ
CH2 = R // N_DEV


def _rope_tables(HD):
    lane = np.arange(HD)
    inv = 10000.0 ** (-(2.0 * ((lane % DH) // 2)) / DH)
    ang = np.arange(SQ)[:, None] * inv[None, :]
    cos = np.tile(np.cos(ang), (2 * B, 1)).astype(np.float32)
    sin = np.tile(np.sin(ang), (2 * B, 1)).astype(np.float32)
    perm = np.zeros((HD, HD), np.float32)
    perm[np.arange(1, HD, 2), np.arange(0, HD, 2)] = -1.0
    perm[np.arange(0, HD, 2), np.arange(1, HD, 2)] = 1.0
    return cos, sin, perm


def kernel(x, Wq, Wk, Wv, Wo):
    HL = Wq.shape[1] // DH
    HD = HL * DH
    cos_np, sin_np, perm_np = _rope_tables(HD)

    def body(x_ref, wq_ref, wk_ref, wv_ref, wo_ref, cos_ref, sin_ref,
             perm_ref, out_ref,
             pbuf_ref, rbuf1_ref, rbuf2_ref,
             s1_sems, r1_sems, s2_sems, r2_sems):
        my = lax.axis_index("i")

        barrier = pltpu.get_barrier_semaphore()
        for t in range(N_DEV):
            @pl.when(t != my)
            def _(t=t):
                pl.semaphore_signal(barrier, inc=1, device_id=(t,),
                                    device_id_type=pl.DeviceIdType.MESH)

        W = jnp.concatenate([wq_ref[:, :], wk_ref[:, :], wv_ref[:, :]],
                            axis=1)
        xb = x_ref[:, :, :].reshape(R, D)
        qkv = jnp.dot(xb, W, preferred_element_type=jnp.float32)
        qk = jnp.concatenate([qkv[:, :HD], qkv[:, HD:2 * HD]],
                             axis=0)
        qk_r = jnp.dot(qk.astype(jnp.bfloat16), perm_ref[:, :],
                       preferred_element_type=jnp.float32)
        qk16 = (qk * cos_ref[:, :] + qk_r * sin_ref[:, :]).astype(jnp.bfloat16)
        q = qk16[:R, :]
        k = qk16[R:, :]
        v = qkv[:, 2 * HD:].astype(jnp.bfloat16)

        ctx_rows = []
        for b in range(B):
            r = slice(b * SQ, (b + 1) * SQ)
            ctxs = []
            for h in range(HL):
                c = slice(h * DH, (h + 1) * DH)
                s = jnp.dot(q[r, c], k[r, c].T,
                            preferred_element_type=jnp.float32) * 0.125
                w = jnp.exp(s)
                w = (w / jnp.sum(w, axis=-1, keepdims=True)
                     ).astype(jnp.bfloat16)
                ctxs.append(jnp.dot(w, v[r, c],
                                    preferred_element_type=jnp.float32))
            ctx_rows.append(jnp.concatenate(ctxs, axis=1))
        ctx = jnp.concatenate(ctx_rows, axis=0)
        pb = jnp.dot(ctx.astype(jnp.bfloat16), wo_ref[:, :],
                     preferred_element_type=jnp.float32)
        pbuf_ref[:, :] = pb.astype(jnp.bfloat16)
        pl.semaphore_wait(barrier, N_DEV - 1)

        for j in range(N_DEV):
            @pl.when(j != my)
            def _(j=j):
                rdma = pltpu.make_async_remote_copy(
                    src_ref=pbuf_ref.at[pl.ds(j * CH2, CH2), :],
                    dst_ref=rbuf1_ref.at[my],
                    send_sem=s1_sems.at[j],
                    recv_sem=r1_sems.at[my],
                    device_id=(j,),
                    device_id_type=pl.DeviceIdType.MESH,
                )
                rdma.start()
        rbuf1_ref[my] = pbuf_ref[pl.ds(my * CH2, CH2), :]
        for s in range(N_DEV):
            @pl.when(s != my)
            def _(s=s):
                recv = pltpu.make_async_remote_copy(
                    src_ref=rbuf1_ref.at[s],
                    dst_ref=rbuf1_ref.at[s],
                    send_sem=s1_sems.at[s],
                    recv_sem=r1_sems.at[s],
                    device_id=(0,),
                    device_id_type=pl.DeviceIdType.MESH,
                )
                recv.wait_recv()
        acc = rbuf1_ref[0, :, :].astype(jnp.float32)
        for s in range(1, N_DEV):
            acc = acc + rbuf1_ref[s, :, :].astype(jnp.float32)
        rbuf2_ref[my] = acc.astype(jnp.bfloat16)

        for t in range(N_DEV):
            @pl.when(t != my)
            def _(t=t):
                rdma = pltpu.make_async_remote_copy(
                    src_ref=rbuf2_ref.at[my],
                    dst_ref=rbuf2_ref.at[my],
                    send_sem=s2_sems.at[t],
                    recv_sem=r2_sems.at[my],
                    device_id=(t,),
                    device_id_type=pl.DeviceIdType.MESH,
                )
                rdma.start()
        for s in range(N_DEV):
            @pl.when(s != my)
            def _(s=s):
                recv = pltpu.make_async_remote_copy(
                    src_ref=rbuf2_ref.at[s],
                    dst_ref=rbuf2_ref.at[s],
                    send_sem=s2_sems.at[s],
                    recv_sem=r2_sems.at[s],
                    device_id=(0,),
                    device_id_type=pl.DeviceIdType.MESH,
                )
                recv.wait_recv()
            out_ref[s // 4, (s % 4) * CH2:((s % 4) + 1) * CH2, :] = \
                rbuf2_ref[s, :, :].astype(jnp.float32)

        for t in range(N_DEV):
            @pl.when(t != my)
            def _(t=t):
                w1 = pltpu.make_async_remote_copy(
                    src_ref=rbuf1_ref.at[t], dst_ref=rbuf1_ref.at[t],
                    send_sem=s1_sems.at[t], recv_sem=r1_sems.at[t],
                    device_id=(0,), device_id_type=pl.DeviceIdType.MESH,
                )
                w1.wait_send()
                w2 = pltpu.make_async_remote_copy(
                    src_ref=rbuf2_ref.at[t], dst_ref=rbuf2_ref.at[t],
                    send_sem=s2_sems.at[t], recv_sem=r2_sems.at[t],
                    device_id=(0,), device_id_type=pl.DeviceIdType.MESH,
                )
                w2.wait_send()

    return pl.pallas_call(
        body,
        out_shape=jax.ShapeDtypeStruct((B, SQ, D), jnp.float32),
        in_specs=[pl.BlockSpec(memory_space=pltpu.VMEM)] * 8,
        out_specs=pl.BlockSpec(memory_space=pltpu.VMEM),
        scratch_shapes=[
            pltpu.VMEM((R, D), jnp.bfloat16),
            pltpu.VMEM((N_DEV, CH2, D), jnp.bfloat16),
            pltpu.VMEM((N_DEV, CH2, D), jnp.bfloat16),
            pltpu.SemaphoreType.DMA((N_DEV,)),
            pltpu.SemaphoreType.DMA((N_DEV,)),
            pltpu.SemaphoreType.DMA((N_DEV,)),
            pltpu.SemaphoreType.DMA((N_DEV,)),
        ],
        compiler_params=pltpu.CompilerParams(collective_id=0),
    )(x.astype(jnp.bfloat16), Wq.astype(jnp.bfloat16),
      Wk.astype(jnp.bfloat16), Wv.astype(jnp.bfloat16),
      Wo.astype(jnp.bfloat16),
      jnp.asarray(cos_np), jnp.asarray(sin_np),
      jnp.asarray(perm_np).astype(jnp.bfloat16))
